# Initial kernel scaffold; baseline (speedup 1.0000x reference)
import jax
import jax.numpy as jnp
from jax.experimental import pallas as pl


def kernel(x, edge_index, feat_author, feat_paper, feat_term, feat_conf, Wt, bt, Wenc, benc, Watt, batt, Wc, bc):
    raise NotImplementedError("write your pallas kernel here")


# speedup vs baseline: 1.3669x; 1.3669x over previous
import jax
import jax.numpy as jnp
from jax.experimental import pallas as pl

_COUNTS = (4000, 4000, 1500, 500)
_D = 128
_M = 6
_L = 2
_N = sum(_COUNTS)
_B = 1000


def _magnn_block(x_ref, wts_ref, bt_ref, enc_ref, benc_ref, wv_ref,
                 bs_ref, wc_ref, bc_ref, out_ref):
    i = pl.program_id(0)
    xb = x_ref[...]

    rows = i * _B + jax.lax.broadcasted_iota(jnp.int32, (_B, 1), 0)
    c0 = _COUNTS[0]
    c1 = c0 + _COUNTS[1]
    c2 = c1 + _COUNTS[2]
    m0 = (rows < c0).astype(jnp.float32)
    m1 = ((rows >= c0) & (rows < c1)).astype(jnp.float32)
    m2 = ((rows >= c1) & (rows < c2)).astype(jnp.float32)
    m3 = (rows >= c2).astype(jnp.float32)
    masks = (m0, m1, m2, m3)

    xcat = jnp.concatenate([xb * mt for mt in masks], axis=1)
    h = jnp.dot(xcat, wts_ref[...], preferred_element_type=jnp.float32)
    for t in range(4):
        h = h + masks[t] * bt_ref[t:t + 1, :]

    for l in range(_L):
        enc = enc_ref[l]
        s = jnp.dot(h, wv_ref[l], preferred_element_type=jnp.float32)
        s = s + bs_ref[l]
        s = jnp.where(s >= 0, s, 0.2 * s)
        s = s - jnp.max(s, axis=1, keepdims=True)
        e = jnp.exp(s)
        w = e / jnp.sum(e, axis=1, keepdims=True)
        hcat = jnp.concatenate([h * w[:, m:m + 1] for m in range(_M)],
                               axis=1)
        hb = jnp.dot(hcat, enc, preferred_element_type=jnp.float32)
        for m in range(_M):
            hb = hb + w[:, m:m + 1] * benc_ref[l, m:m + 1, :]
        h = jnp.where(hb > 0, hb, jnp.exp(jnp.minimum(hb, 0.0)) - 1.0)

    out_ref[...] = jnp.dot(h, wc_ref[...],
                           preferred_element_type=jnp.float32) + bc_ref[...]


@jax.jit
def _magnn_forward(xall, wts, bt, encs, benc, wv, bs, wc, bc2):
    grid = (_N // _B,)
    full = lambda shape: pl.BlockSpec(shape, lambda i: (0,) * len(shape))
    return pl.pallas_call(
        _magnn_block,
        grid=grid,
        in_specs=[
            pl.BlockSpec((_B, _D), lambda i: (i, 0)),
            full((4 * _D, _D)),
            full((4, _D)),
            full((_L, _M * _D, _D)),
            full((_L, _M, _D)),
            full((_L, _D, _M)),
            full((_L, 1, _M)),
            full((_D, 4)),
            full((1, 4)),
        ],
        out_specs=pl.BlockSpec((_B, 4), lambda i: (i, 0)),
        out_shape=jax.ShapeDtypeStruct((_N, 4), jnp.float32),
    )(xall, wts, bt, encs, benc, wv, bs, wc, bc2)


def kernel(x, edge_index, feat_author, feat_paper, feat_term, feat_conf,
           Wt, bt, Wenc, benc, Watt, batt, Wc, bc):
    xall = jnp.concatenate([feat_author, feat_paper, feat_term, feat_conf],
                           axis=0)
    wts = Wt.reshape(4 * _D, _D)
    encs = Wenc.reshape(_L, _M * _D, _D)
    wv = jnp.einsum('lmdh,lh->ldm', Wenc, Watt)
    bs = (jnp.einsum('lmh,lh->lm', benc, Watt)
          + batt[:, None])[:, None, :]
    return _magnn_forward(xall, wts, bt, encs, benc, wv, bs, Wc,
                          bc.reshape(1, 4))


# speedup vs baseline: 1.3696x; 1.0020x over previous
import jax
import jax.numpy as jnp
from jax.experimental import pallas as pl

_COUNTS = (4000, 4000, 1500, 500)
_D = 128
_M = 6
_L = 2
_N = sum(_COUNTS)
_B = 1000


def _magnn_block(x_ref, wts_ref, bts_ref, enc_ref, benc_ref, wv_ref,
                 bs_ref, wc_ref, bc_ref, out_ref):
    i = pl.program_id(0)
    xb = x_ref[...]

    rows = i * _B + jax.lax.broadcasted_iota(jnp.int32, (_B, 1), 0)
    c0 = _COUNTS[0]
    c1 = c0 + _COUNTS[1]
    c2 = c1 + _COUNTS[2]

    h_all = jnp.dot(xb, wts_ref[...], preferred_element_type=jnp.float32)
    h_all = h_all + bts_ref[...]
    h = jnp.where(
        rows < c0, h_all[:, :_D],
        jnp.where(rows < c1, h_all[:, _D:2 * _D],
                  jnp.where(rows < c2, h_all[:, 2 * _D:3 * _D],
                            h_all[:, 3 * _D:])))

    for l in range(_L):
        o = jnp.dot(h, enc_ref[l], preferred_element_type=jnp.float32)
        s = jnp.dot(h, wv_ref[l], preferred_element_type=jnp.float32)
        s = s + bs_ref[l]
        s = jnp.where(s >= 0, s, 0.2 * s)
        s = s - jnp.max(s, axis=1, keepdims=True)
        e = jnp.exp(s)
        w = e / jnp.sum(e, axis=1, keepdims=True)
        hb = jnp.dot(w, benc_ref[l], preferred_element_type=jnp.float32)
        for m in range(_M):
            hb = hb + w[:, m:m + 1] * o[:, m * _D:(m + 1) * _D]
        h = jnp.where(hb > 0, hb, jnp.exp(jnp.minimum(hb, 0.0)) - 1.0)

    out_ref[...] = jnp.dot(h, wc_ref[...],
                           preferred_element_type=jnp.float32) + bc_ref[...]


@jax.jit
def _magnn_forward(xall, wts, bts, encs, benc, wv, bs, wc, bc2):
    grid = (_N // _B,)
    full = lambda shape: pl.BlockSpec(shape, lambda i: (0,) * len(shape))
    return pl.pallas_call(
        _magnn_block,
        grid=grid,
        in_specs=[
            pl.BlockSpec((_B, _D), lambda i: (i, 0)),
            full((_D, 4 * _D)),
            full((1, 4 * _D)),
            full((_L, _D, _M * _D)),
            full((_L, _M, _D)),
            full((_L, _D, _M)),
            full((_L, 1, _M)),
            full((_D, 4)),
            full((1, 4)),
        ],
        out_specs=pl.BlockSpec((_B, 4), lambda i: (i, 0)),
        out_shape=jax.ShapeDtypeStruct((_N, 4), jnp.float32),
    )(xall, wts, bts, encs, benc, wv, bs, wc, bc2)


def kernel(x, edge_index, feat_author, feat_paper, feat_term, feat_conf,
           Wt, bt, Wenc, benc, Watt, batt, Wc, bc):
    xall = jnp.concatenate([feat_author, feat_paper, feat_term, feat_conf],
                           axis=0)
    wts = Wt.transpose(1, 0, 2).reshape(_D, 4 * _D)
    bts = bt.reshape(1, 4 * _D)
    encs = Wenc.transpose(0, 2, 1, 3).reshape(_L, _D, _M * _D)
    wv = jnp.einsum('lmdh,lh->ldm', Wenc, Watt)
    bs = (jnp.einsum('lmh,lh->lm', benc, Watt)
          + batt[:, None])[:, None, :]
    return _magnn_forward(xall, wts, bts, encs, benc, wv, bs, Wc,
                          bc.reshape(1, 4))


# speedup vs baseline: 1.4990x; 1.0944x over previous
import jax
import jax.numpy as jnp
from jax.experimental import pallas as pl

_COUNTS = (4000, 4000, 1500, 500)
_D = 128
_M = 6
_L = 2
_N = sum(_COUNTS)
_B = 2000


def _magnn_block(x_ref, wts_ref, bts_ref, enc_ref, benc_ref, wv_ref,
                 bs_ref, wc_ref, bc_ref, out_ref):
    i = pl.program_id(0)
    xb = x_ref[...]

    rows = i * _B + jax.lax.broadcasted_iota(jnp.int32, (_B, 1), 0)
    c0 = _COUNTS[0]
    c1 = c0 + _COUNTS[1]
    c2 = c1 + _COUNTS[2]

    h_all = jnp.dot(xb, wts_ref[...], preferred_element_type=jnp.float32)
    h_all = h_all + bts_ref[...]
    h = jnp.where(
        rows < c0, h_all[:, :_D],
        jnp.where(rows < c1, h_all[:, _D:2 * _D],
                  jnp.where(rows < c2, h_all[:, 2 * _D:3 * _D],
                            h_all[:, 3 * _D:])))

    for l in range(_L):
        o = jnp.dot(h, enc_ref[l], preferred_element_type=jnp.float32)
        s = jnp.dot(h, wv_ref[l], preferred_element_type=jnp.float32)
        s = s + bs_ref[l]
        s = jnp.where(s >= 0, s, 0.2 * s)
        s = s - jnp.max(s, axis=1, keepdims=True)
        e = jnp.exp(s)
        w = e / jnp.sum(e, axis=1, keepdims=True)
        hb = jnp.dot(w, benc_ref[l], preferred_element_type=jnp.float32)
        for m in range(_M):
            hb = hb + w[:, m:m + 1] * o[:, m * _D:(m + 1) * _D]
        h = jnp.where(hb > 0, hb, jnp.exp(jnp.minimum(hb, 0.0)) - 1.0)

    out_ref[...] = jnp.dot(h, wc_ref[...],
                           preferred_element_type=jnp.float32) + bc_ref[...]


@jax.jit
def _magnn_forward(xall, wts, bts, encs, benc, wv, bs, wc, bc2):
    grid = (_N // _B,)
    full = lambda shape: pl.BlockSpec(shape, lambda i: (0,) * len(shape))
    return pl.pallas_call(
        _magnn_block,
        grid=grid,
        in_specs=[
            pl.BlockSpec((_B, _D), lambda i: (i, 0)),
            full((_D, 4 * _D)),
            full((1, 4 * _D)),
            full((_L, _D, _M * _D)),
            full((_L, _M, _D)),
            full((_L, _D, _M)),
            full((_L, 1, _M)),
            full((_D, 4)),
            full((1, 4)),
        ],
        out_specs=pl.BlockSpec((_B, 4), lambda i: (i, 0)),
        out_shape=jax.ShapeDtypeStruct((_N, 4), jnp.float32),
    )(xall, wts, bts, encs, benc, wv, bs, wc, bc2)


def kernel(x, edge_index, feat_author, feat_paper, feat_term, feat_conf,
           Wt, bt, Wenc, benc, Watt, batt, Wc, bc):
    xall = jnp.concatenate([feat_author, feat_paper, feat_term, feat_conf],
                           axis=0)
    wts = Wt.transpose(1, 0, 2).reshape(_D, 4 * _D)
    bts = bt.reshape(1, 4 * _D)
    encs = Wenc.transpose(0, 2, 1, 3).reshape(_L, _D, _M * _D)
    wv = jnp.einsum('lmdh,lh->ldm', Wenc, Watt)
    bs = (jnp.einsum('lmh,lh->lm', benc, Watt)
          + batt[:, None])[:, None, :]
    return _magnn_forward(xall, wts, bts, encs, benc, wv, bs, Wc,
                          bc.reshape(1, 4))
